# Initial kernel scaffold; baseline (speedup 1.0000x reference)
import jax
import jax.numpy as jnp
from jax.experimental import pallas as pl


def kernel(x, table):
    raise NotImplementedError("write your pallas kernel here")


# speedup vs baseline: 1.2887x; 1.2887x over previous
import functools

import jax
import jax.numpy as jnp
from jax import lax
from jax.experimental import pallas as pl
from jax.experimental.pallas import tpu as pltpu
from jax.experimental.pallas import tpu_sc as plsc

_NC = 2
_NS = 16
_NW = _NC * _NS
_IW = 128
_K = 8


@functools.partial(jax.jit, static_argnames=("rows_per_w", "d"))
def _sc_gather(idx2d, table, rows_per_w, d):
    n_rows = idx2d.shape[0]
    mesh = plsc.VectorSubcoreMesh(
        core_axis_name="c", subcore_axis_name="s",
        num_cores=_NC, num_subcores=_NS)

    @functools.partial(
        pl.kernel,
        out_type=jax.ShapeDtypeStruct((n_rows, _IW, d), jnp.float32),
        mesh=mesh,
        scratch_types=[
            pltpu.VMEM((_K, _IW), jnp.int32),
            pltpu.VMEM((_K, _IW, d), jnp.float32),
            pltpu.SemaphoreType.DMA,
        ],
        compiler_params=pltpu.CompilerParams(use_tc_tiling_on_sc=False),
    )
    def k(idx_hbm, table_hbm, out_hbm, idx_v, rows_v, sem):
        wid = lax.axis_index("s") * _NC + lax.axis_index("c")
        base = wid * rows_per_w

        def body(g, carry):
            r0 = base + g * _K
            pltpu.sync_copy(idx_hbm.at[pl.ds(r0, _K)], idx_v)
            descs = [
                pltpu.async_copy(table_hbm.at[idx_v.at[j]], rows_v.at[j], sem)
                for j in range(_K)
            ]
            for dsc in descs:
                dsc.wait()
            pltpu.sync_copy(rows_v, out_hbm.at[pl.ds(r0, _K)])
            return carry

        lax.fori_loop(0, rows_per_w // _K, body, 0)

    return k(idx2d, table)


def kernel(x, table):
    b, h = x.shape
    d = table.shape[1]
    idx = x.reshape(-1).astype(jnp.int32)
    n = idx.shape[0]
    assert n % (_IW * _NW * _K) == 0
    idx2d = idx.reshape(-1, _IW)
    rows_per_w = idx2d.shape[0] // _NW
    out = _sc_gather(idx2d, table, rows_per_w, d)
    return out.reshape(b, h, d)
